# Initial kernel scaffold; baseline (speedup 1.0000x reference)
import jax
import jax.numpy as jnp
from jax.experimental import pallas as pl


def kernel(x, edge_index, edge_type, weight_matrices, bias):
    raise NotImplementedError("write your pallas kernel here")


# speedup vs baseline: 6.9307x; 6.9307x over previous
import functools

import jax
import jax.numpy as jnp
from jax import lax
from jax.experimental import pallas as pl
from jax.experimental.pallas import tpu as pltpu
from jax.experimental.pallas import tpu_sc as plsc

N_NODES = 10000
N_EDGES = 320000
D = 128
NREL = 4

NTILES = 16
NCORES = 2
NPAD = 10240
CHUNK = 2560
NPASS = 2
EPT = N_EDGES // NTILES
SB = 2000
NSB = EPT // SB
BE = 80
NB = SB // BE
DA = 144
ACC_ROWS = NREL * CHUNK + 128
DUMP = NREL * CHUNK
ZROWS = ACC_ROWS // NTILES
RPT = NREL * CHUNK // NTILES


def _sc_body(x_hbm, src_hbm, dst_hbm, typ_hbm, zrow_hbm,
             sums_hbm,
             src_sb, dst_sb, typ_sb, sidx_sb, aidx_sb, rows_v,
             acc_sh):
    core = lax.axis_index("c")
    sub = lax.axis_index("s")

    for p in range(NPASS):
        chunk_id = NPASS * core + p
        lo = chunk_id * CHUNK

        pltpu.sync_copy(zrow_hbm, acc_sh.at[pl.ds(sub * ZROWS, ZROWS)])
        plsc.subcore_barrier()

        for sb in range(NSB):
            base = sub * EPT + sb * SB
            pltpu.sync_copy(src_hbm.at[pl.ds(base, SB)], src_sb)
            pltpu.sync_copy(dst_hbm.at[pl.ds(base, SB)], dst_sb)
            pltpu.sync_copy(typ_hbm.at[pl.ds(base, SB)], typ_sb)

            @pl.loop(0, NB)
            def _block(b):
                e0 = b * BE
                for g in range(BE // 16):
                    off = e0 + g * 16
                    d16 = dst_sb[pl.ds(off, 16)]
                    t16 = typ_sb[pl.ds(off, 16)]
                    s16 = src_sb[pl.ds(off, 16)]
                    m = (d16 >= lo) & (d16 < lo + CHUNK)
                    loc = t16 * CHUNK + (d16 - lo)
                    a16 = jnp.where(m, loc, DUMP)
                    aidx_sb[b, pl.ds(g * 16, 16)] = a16
                    sidx_sb[b, pl.ds(g * 16, 16)] = s16
                pltpu.sync_copy(x_hbm.at[sidx_sb.at[b]], rows_v)
                pltpu.sync_copy(rows_v, acc_sh.at[aidx_sb.at[b]], add=True)

        plsc.subcore_barrier()

        rel = sub // (CHUNK // RPT)
        q = sub % (CHUNK // RPT)
        row0 = rel * NPAD + lo + q * RPT
        pltpu.sync_copy(acc_sh.at[pl.ds(sub * RPT, RPT)],
                        sums_hbm.at[pl.ds(row0, RPT)])
        plsc.subcore_barrier()


def _sc_aggregate(xa, src, dst, etype):
    zrow = jnp.zeros((ZROWS, DA), jnp.float32)
    mesh = plsc.VectorSubcoreMesh(core_axis_name="c", subcore_axis_name="s")
    f = pl.kernel(
        _sc_body,
        out_type=jax.ShapeDtypeStruct((NREL * NPAD, DA), jnp.float32),
        mesh=mesh,
        compiler_params=pltpu.CompilerParams(use_tc_tiling_on_sc=False),
        scratch_types=[
            pltpu.VMEM((SB,), jnp.int32),
            pltpu.VMEM((SB,), jnp.int32),
            pltpu.VMEM((SB,), jnp.int32),
            pltpu.VMEM((NB, BE), jnp.int32),
            pltpu.VMEM((NB, BE), jnp.int32),
            pltpu.VMEM((BE, DA), jnp.float32),
            pltpu.VMEM_SHARED((ACC_ROWS, DA), jnp.float32),
        ],
    )
    return f(xa, src, dst, etype, zrow)


BM = 512


def _tc_body(s_ref, w_ref, b_ref, o_ref):
    acc = jnp.zeros((BM, D), jnp.float32)
    for r in range(NREL):
        blk = s_ref[r]
        s = blk[:, :D]
        cr = blk[:, D:D + 1]
        t = jnp.dot(s, w_ref[r], precision=lax.Precision.HIGHEST,
                    preferred_element_type=jnp.float32)
        acc = acc + jnp.where(cr > 0.0, t / jnp.maximum(cr, 1.0),
                              jnp.zeros_like(t))
    o_ref[...] = acc + b_ref[...]


def _tc_combine(sums, weights, bias):
    grid = (NPAD // BM,)
    return pl.pallas_call(
        _tc_body,
        grid=grid,
        in_specs=[
            pl.BlockSpec((NREL, BM, DA), lambda i: (0, i, 0)),
            pl.BlockSpec((NREL, D, D), lambda i: (0, 0, 0)),
            pl.BlockSpec((1, D), lambda i: (0, 0)),
        ],
        out_specs=pl.BlockSpec((BM, D), lambda i: (i, 0)),
        out_shape=jax.ShapeDtypeStruct((NPAD, D), jnp.float32),
    )(sums, weights, bias)


@jax.jit
def kernel(x, edge_index, edge_type, weight_matrices, bias):
    src = edge_index[0]
    dst = edge_index[1]
    ones_col = jnp.ones((N_NODES, 1), jnp.float32)
    pad = jnp.zeros((N_NODES, DA - D - 1), jnp.float32)
    xa = jnp.concatenate([x, ones_col, pad], axis=1)
    sums = _sc_aggregate(xa, src, dst, edge_type)
    sums = sums.reshape(NREL, NPAD, DA)
    out = _tc_combine(sums, weight_matrices, bias.reshape(1, D))
    return out[:N_NODES]


# speedup vs baseline: 8.3385x; 1.2031x over previous
import functools

import jax
import jax.numpy as jnp
from jax import lax
from jax.experimental import pallas as pl
from jax.experimental.pallas import tpu as pltpu
from jax.experimental.pallas import tpu_sc as plsc

N_NODES = 10000
N_EDGES = 320000
D = 128
NREL = 4

NTILES = 16
NCORES = 2
NPAD = 10240
CHUNK = 2560
NPASS = 2
EPT = N_EDGES // NTILES
SB = 2000
NSB = EPT // SB
BE = 80
NB = SB // BE
DA = 144
ACC_ROWS = NREL * CHUNK + 128
DUMP = NREL * CHUNK
ZROWS = ACC_ROWS // NTILES
RPT = NREL * CHUNK // NTILES


def _sc_body(x_hbm, src_hbm, dst_hbm, typ_hbm, zrow_hbm,
             sums_hbm,
             src_sb, dst_sb, typ_sb, srow, arow, rows0, rows1,
             gs0, gs1, ss0, ss1,
             acc_sh):
    core = lax.axis_index("c")
    sub = lax.axis_index("s")
    rows = (rows0, rows1)
    gsem = (gs0, gs1)
    ssem = (ss0, ss1)

    for p in range(NPASS):
        chunk_id = NPASS * core + p
        lo = chunk_id * CHUNK

        pltpu.sync_copy(zrow_hbm, acc_sh.at[pl.ds(sub * ZROWS, ZROWS)])
        plsc.subcore_barrier()

        @pl.loop(0, NSB)
        def _superblock(sb):
            base = sub * EPT + sb * SB
            pltpu.sync_copy(src_hbm.at[pl.ds(base, SB)], src_sb)
            pltpu.sync_copy(dst_hbm.at[pl.ds(base, SB)], dst_sb)
            pltpu.sync_copy(typ_hbm.at[pl.ds(base, SB)], typ_sb)

            def prep(b):
                slot = b % 2
                for g in range(BE // 16):
                    off = b * BE + g * 16
                    d16 = dst_sb[pl.ds(off, 16)]
                    t16 = typ_sb[pl.ds(off, 16)]
                    s16 = src_sb[pl.ds(off, 16)]
                    m = (d16 >= lo) & (d16 < lo + CHUNK)
                    a16 = jnp.where(m, t16 * CHUNK + (d16 - lo), DUMP)
                    srow[slot, pl.ds(g * 16, 16)] = s16
                    arow[slot, pl.ds(g * 16, 16)] = a16

            gds = [None] * NB
            sds = [None] * NB
            prep(0)
            gds[0] = pltpu.async_copy(x_hbm.at[srow.at[0]], rows[0], gsem[0])
            for b in range(NB):
                par = b % 2
                nxt = 1 - par
                if b + 1 < NB:
                    if b >= 1:
                        sds[b - 1].wait()
                    prep(b + 1)
                    gds[b + 1] = pltpu.async_copy(
                        x_hbm.at[srow.at[nxt]], rows[nxt], gsem[nxt])
                gds[b].wait()
                sds[b] = pltpu.async_copy(
                    rows[par], acc_sh.at[arow.at[par]], ssem[par], add=True)
            sds[NB - 2].wait()
            sds[NB - 1].wait()

        plsc.subcore_barrier()

        rel = sub // (CHUNK // RPT)
        q = sub % (CHUNK // RPT)
        row0 = rel * NPAD + lo + q * RPT
        pltpu.sync_copy(acc_sh.at[pl.ds(sub * RPT, RPT)],
                        sums_hbm.at[pl.ds(row0, RPT)])
        plsc.subcore_barrier()


def _sc_aggregate(xa, src, dst, etype):
    zrow = jnp.zeros((ZROWS, DA), jnp.float32)
    mesh = plsc.VectorSubcoreMesh(core_axis_name="c", subcore_axis_name="s")
    f = pl.kernel(
        _sc_body,
        out_type=jax.ShapeDtypeStruct((NREL * NPAD, DA), jnp.float32),
        mesh=mesh,
        compiler_params=pltpu.CompilerParams(use_tc_tiling_on_sc=False),
        scratch_types=[
            pltpu.VMEM((SB,), jnp.int32),
            pltpu.VMEM((SB,), jnp.int32),
            pltpu.VMEM((SB,), jnp.int32),
            pltpu.VMEM((2, BE), jnp.int32),
            pltpu.VMEM((2, BE), jnp.int32),
            pltpu.VMEM((BE, DA), jnp.float32),
            pltpu.VMEM((BE, DA), jnp.float32),
            pltpu.SemaphoreType.DMA,
            pltpu.SemaphoreType.DMA,
            pltpu.SemaphoreType.DMA,
            pltpu.SemaphoreType.DMA,
            pltpu.VMEM_SHARED((ACC_ROWS, DA), jnp.float32),
        ],
    )
    return f(xa, src, dst, etype, zrow)


BM = 512


def _tc_body(s_ref, w_ref, b_ref, o_ref):
    acc = jnp.zeros((BM, D), jnp.float32)
    for r in range(NREL):
        blk = s_ref[r]
        s = blk[:, :D]
        cr = blk[:, D:D + 1]
        t = jnp.dot(s, w_ref[r], precision=lax.Precision.HIGHEST,
                    preferred_element_type=jnp.float32)
        acc = acc + jnp.where(cr > 0.0, t / jnp.maximum(cr, 1.0),
                              jnp.zeros_like(t))
    o_ref[...] = acc + b_ref[...]


def _tc_combine(sums, weights, bias):
    grid = (NPAD // BM,)
    return pl.pallas_call(
        _tc_body,
        grid=grid,
        in_specs=[
            pl.BlockSpec((NREL, BM, DA), lambda i: (0, i, 0)),
            pl.BlockSpec((NREL, D, D), lambda i: (0, 0, 0)),
            pl.BlockSpec((1, D), lambda i: (0, 0)),
        ],
        out_specs=pl.BlockSpec((BM, D), lambda i: (i, 0)),
        out_shape=jax.ShapeDtypeStruct((NPAD, D), jnp.float32),
    )(sums, weights, bias)


@jax.jit
def kernel(x, edge_index, edge_type, weight_matrices, bias):
    src = edge_index[0]
    dst = edge_index[1]
    ones_col = jnp.ones((N_NODES, 1), jnp.float32)
    pad = jnp.zeros((N_NODES, DA - D - 1), jnp.float32)
    xa = jnp.concatenate([x, ones_col, pad], axis=1)
    sums = _sc_aggregate(xa, src, dst, edge_type)
    sums = sums.reshape(NREL, NPAD, DA)
    out = _tc_combine(sums, weight_matrices, bias.reshape(1, D))
    return out[:N_NODES]


# speedup vs baseline: 18.5784x; 2.2280x over previous
import functools

import jax
import jax.numpy as jnp
from jax import lax
from jax.experimental import pallas as pl
from jax.experimental.pallas import tpu as pltpu
from jax.experimental.pallas import tpu_sc as plsc

N_NODES = 10000
N_EDGES = 320000
D = 128
NREL = 4

NTILES = 16
NCORES = 2
NPAD = 10240
CHUNK = 5120
EPT = N_EDGES // NTILES
SB = 2000
NSB = EPT // SB
BE = 80
NB = SB // BE
DA = 160
ACC_ROWS = NREL * CHUNK + 128
DUMP = NREL * CHUNK
ZROWS = ACC_ROWS // NTILES
RPT = NREL * CHUNK // NTILES


def _sc_body(x_hbm, src_hbm, dst_hbm, typ_hbm, zrow_hbm,
             sums_hbm,
             src_sb, dst_sb, typ_sb, srow, arow, rows0, rows1,
             gs0, gs1, ss0, ss1,
             acc_sh):
    core = lax.axis_index("c")
    sub = lax.axis_index("s")
    rows = (rows0, rows1)
    gsem = (gs0, gs1)
    ssem = (ss0, ss1)

    if True:
        lo = core * CHUNK

        pltpu.sync_copy(zrow_hbm, acc_sh.at[pl.ds(sub * ZROWS, ZROWS)])
        plsc.subcore_barrier()

        @pl.loop(0, NSB)
        def _superblock(sb):
            base = sub * EPT + sb * SB
            pltpu.sync_copy(src_hbm.at[pl.ds(base, SB)], src_sb)
            pltpu.sync_copy(dst_hbm.at[pl.ds(base, SB)], dst_sb)
            pltpu.sync_copy(typ_hbm.at[pl.ds(base, SB)], typ_sb)

            def prep(b):
                slot = b % 2
                for g in range(BE // 16):
                    off = b * BE + g * 16
                    d16 = dst_sb[pl.ds(off, 16)]
                    t16 = typ_sb[pl.ds(off, 16)]
                    s16 = src_sb[pl.ds(off, 16)]
                    m = (d16 >= lo) & (d16 < lo + CHUNK)
                    a16 = jnp.where(m, t16 * CHUNK + (d16 - lo), DUMP)
                    srow[slot, pl.ds(g * 16, 16)] = s16
                    arow[slot, pl.ds(g * 16, 16)] = a16

            gds = [None] * NB
            sds = [None] * NB
            prep(0)
            gds[0] = pltpu.async_copy(x_hbm.at[srow.at[0]], rows[0], gsem[0])
            for b in range(NB):
                par = b % 2
                nxt = 1 - par
                if b + 1 < NB:
                    if b >= 1:
                        sds[b - 1].wait()
                    prep(b + 1)
                    gds[b + 1] = pltpu.async_copy(
                        x_hbm.at[srow.at[nxt]], rows[nxt], gsem[nxt])
                gds[b].wait()
                sds[b] = pltpu.async_copy(
                    rows[par], acc_sh.at[arow.at[par]], ssem[par], add=True)
            sds[NB - 2].wait()
            sds[NB - 1].wait()

        plsc.subcore_barrier()

        rel = sub // (CHUNK // RPT)
        q = sub % (CHUNK // RPT)
        row0 = rel * NPAD + lo + q * RPT
        pltpu.sync_copy(acc_sh.at[pl.ds(sub * RPT, RPT)],
                        sums_hbm.at[pl.ds(row0, RPT)])


def _sc_aggregate(xa, src, dst, etype):
    zrow = jnp.zeros((ZROWS, DA), jnp.bfloat16)
    mesh = plsc.VectorSubcoreMesh(core_axis_name="c", subcore_axis_name="s")
    f = pl.kernel(
        _sc_body,
        out_type=jax.ShapeDtypeStruct((NREL * NPAD, DA), jnp.bfloat16),
        mesh=mesh,
        compiler_params=pltpu.CompilerParams(use_tc_tiling_on_sc=False),
        scratch_types=[
            pltpu.VMEM((SB,), jnp.int32),
            pltpu.VMEM((SB,), jnp.int32),
            pltpu.VMEM((SB,), jnp.int32),
            pltpu.VMEM((2, BE), jnp.int32),
            pltpu.VMEM((2, BE), jnp.int32),
            pltpu.VMEM((BE, DA), jnp.bfloat16),
            pltpu.VMEM((BE, DA), jnp.bfloat16),
            pltpu.SemaphoreType.DMA,
            pltpu.SemaphoreType.DMA,
            pltpu.SemaphoreType.DMA,
            pltpu.SemaphoreType.DMA,
            pltpu.VMEM_SHARED((ACC_ROWS, DA), jnp.bfloat16),
        ],
    )
    return f(xa, src, dst, etype, zrow)


BM = 512


def _tc_body(s_ref, w_ref, b_ref, o_ref):
    acc = jnp.zeros((BM, D), jnp.float32)
    for r in range(NREL):
        blk = s_ref[r].astype(jnp.float32)
        s = blk[:, :D]
        cr = blk[:, D:D + 1]
        t = jnp.dot(s, w_ref[r], precision=lax.Precision.HIGHEST,
                    preferred_element_type=jnp.float32)
        acc = acc + jnp.where(cr > 0.0, t / jnp.maximum(cr, 1.0),
                              jnp.zeros_like(t))
    o_ref[...] = acc + b_ref[...]


def _tc_combine(sums, weights, bias):
    grid = (NPAD // BM,)
    return pl.pallas_call(
        _tc_body,
        grid=grid,
        in_specs=[
            pl.BlockSpec((NREL, BM, DA), lambda i: (0, i, 0)),
            pl.BlockSpec((NREL, D, D), lambda i: (0, 0, 0)),
            pl.BlockSpec((1, D), lambda i: (0, 0)),
        ],
        out_specs=pl.BlockSpec((BM, D), lambda i: (i, 0)),
        out_shape=jax.ShapeDtypeStruct((NPAD, D), jnp.float32),
    )(sums, weights, bias)


@jax.jit
def kernel(x, edge_index, edge_type, weight_matrices, bias):
    src = edge_index[0]
    dst = edge_index[1]
    ones_col = jnp.ones((N_NODES, 1), jnp.float32)
    pad = jnp.zeros((N_NODES, DA - D - 1), jnp.float32)
    xa = jnp.concatenate([x, ones_col, pad], axis=1).astype(jnp.bfloat16)
    sums = _sc_aggregate(xa, src, dst, edge_type)
    sums = sums.reshape(NREL, NPAD, DA)
    out = _tc_combine(sums, weight_matrices, bias.reshape(1, D))
    return out[:N_NODES]


# speedup vs baseline: 18.6694x; 1.0049x over previous
import functools

import jax
import jax.numpy as jnp
from jax import lax
from jax.experimental import pallas as pl
from jax.experimental.pallas import tpu as pltpu
from jax.experimental.pallas import tpu_sc as plsc

N_NODES = 10000
N_EDGES = 320000
D = 128
NREL = 4

NTILES = 16
NCORES = 2
NPAD = 10240
CHUNK = 5120
EPT = N_EDGES // NTILES
SB = 2000
NSB = EPT // SB
BE = 80
NB = SB // BE
DA = 160
ACC_ROWS = NREL * CHUNK + 128
DUMP = NREL * CHUNK
ZROWS = ACC_ROWS // NTILES
RPT = NREL * CHUNK // NTILES


def _sc_body(x_hbm, src_hbm, dst_hbm, typ_hbm, zrow_hbm,
             sums_hbm,
             src_sb, dst_sb, typ_sb, srow, arow, rows0, rows1,
             gs0, gs1, ss0, ss1,
             acc_sh):
    core = lax.axis_index("c")
    sub = lax.axis_index("s")
    rows = (rows0, rows1)
    gsem = (gs0, gs1)
    ssem = (ss0, ss1)

    if True:
        lo = core * CHUNK

        pltpu.sync_copy(zrow_hbm, acc_sh.at[pl.ds(sub * ZROWS, ZROWS)])
        plsc.subcore_barrier()

        @pl.loop(0, NSB)
        def _superblock(sb):
            base = sub * EPT + sb * SB
            pltpu.sync_copy(src_hbm.at[pl.ds(base, SB)], src_sb)
            pltpu.sync_copy(dst_hbm.at[pl.ds(base, SB)], dst_sb)
            pltpu.sync_copy(typ_hbm.at[pl.ds(base, SB)], typ_sb)

            def prep(b):
                slot = b % 2
                for g in range(BE // 16):
                    off = b * BE + g * 16
                    d16 = dst_sb[pl.ds(off, 16)]
                    t16 = typ_sb[pl.ds(off, 16)]
                    s16 = src_sb[pl.ds(off, 16)]
                    m = (d16 >= lo) & (d16 < lo + CHUNK)
                    a16 = jnp.where(m, t16 * CHUNK + (d16 - lo), DUMP)
                    srow[slot, pl.ds(g * 16, 16)] = s16
                    arow[slot, pl.ds(g * 16, 16)] = a16

            gds = [None] * NB
            sds = [None] * NB
            prep(0)
            gds[0] = pltpu.async_copy(x_hbm.at[srow.at[0]], rows[0], gsem[0])
            for b in range(NB):
                par = b % 2
                nxt = 1 - par
                if b + 1 < NB:
                    if b >= 1:
                        sds[b - 1].wait()
                    prep(b + 1)
                    gds[b + 1] = pltpu.async_copy(
                        x_hbm.at[srow.at[nxt]], rows[nxt], gsem[nxt])
                gds[b].wait()
                sds[b] = pltpu.async_copy(
                    rows[par], acc_sh.at[arow.at[par]], ssem[par], add=True)
            sds[NB - 2].wait()
            sds[NB - 1].wait()

        plsc.subcore_barrier()

        rel = sub // (CHUNK // RPT)
        q = sub % (CHUNK // RPT)
        row0 = rel * NPAD + lo + q * RPT
        pltpu.sync_copy(acc_sh.at[pl.ds(sub * RPT, RPT)],
                        sums_hbm.at[pl.ds(row0, RPT)])


def _sc_aggregate(xa, src, dst, etype):
    zrow = jnp.zeros((ZROWS, DA), jnp.bfloat16)
    mesh = plsc.VectorSubcoreMesh(core_axis_name="c", subcore_axis_name="s")
    f = pl.kernel(
        _sc_body,
        out_type=jax.ShapeDtypeStruct((NREL * NPAD, DA), jnp.bfloat16),
        mesh=mesh,
        compiler_params=pltpu.CompilerParams(use_tc_tiling_on_sc=False),
        scratch_types=[
            pltpu.VMEM((SB,), jnp.int32),
            pltpu.VMEM((SB,), jnp.int32),
            pltpu.VMEM((SB,), jnp.int32),
            pltpu.VMEM((2, BE), jnp.int32),
            pltpu.VMEM((2, BE), jnp.int32),
            pltpu.VMEM((BE, DA), jnp.bfloat16),
            pltpu.VMEM((BE, DA), jnp.bfloat16),
            pltpu.SemaphoreType.DMA,
            pltpu.SemaphoreType.DMA,
            pltpu.SemaphoreType.DMA,
            pltpu.SemaphoreType.DMA,
            pltpu.VMEM_SHARED((ACC_ROWS, DA), jnp.bfloat16),
        ],
    )
    return f(xa, src, dst, etype, zrow)


BM = 400


def _tc_body(s_ref, w_ref, b_ref, o_ref):
    acc = jnp.zeros((BM, D), jnp.float32)
    for r in range(NREL):
        blk = s_ref[r].astype(jnp.float32)
        s = blk[:, :D]
        cr = blk[:, D:D + 1]
        t = jnp.dot(s, w_ref[r], precision=lax.Precision.HIGHEST,
                    preferred_element_type=jnp.float32)
        acc = acc + jnp.where(cr > 0.0, t / jnp.maximum(cr, 1.0),
                              jnp.zeros_like(t))
    o_ref[...] = acc + b_ref[...]


def _tc_combine(sums, weights, bias):
    grid = (N_NODES // BM,)
    return pl.pallas_call(
        _tc_body,
        grid=grid,
        in_specs=[
            pl.BlockSpec((NREL, BM, DA), lambda i: (0, i, 0)),
            pl.BlockSpec((NREL, D, D), lambda i: (0, 0, 0)),
            pl.BlockSpec((1, D), lambda i: (0, 0)),
        ],
        out_specs=pl.BlockSpec((BM, D), lambda i: (i, 0)),
        out_shape=jax.ShapeDtypeStruct((N_NODES, D), jnp.float32),
    )(sums, weights, bias)


@jax.jit
def kernel(x, edge_index, edge_type, weight_matrices, bias):
    src = edge_index[0]
    dst = edge_index[1]
    ones_col = jnp.ones((N_NODES, 1), jnp.float32)
    pad = jnp.zeros((N_NODES, DA - D - 1), jnp.float32)
    xa = jnp.concatenate([x, ones_col, pad], axis=1).astype(jnp.bfloat16)
    sums = _sc_aggregate(xa, src, dst, edge_type)
    sums = sums.reshape(NREL, NPAD, DA)
    return _tc_combine(sums, weight_matrices, bias.reshape(1, D))


# speedup vs baseline: 18.8692x; 1.0107x over previous
import functools

import jax
import jax.numpy as jnp
from jax import lax
from jax.experimental import pallas as pl
from jax.experimental.pallas import tpu as pltpu
from jax.experimental.pallas import tpu_sc as plsc

N_NODES = 10000
N_EDGES = 320000
D = 128
NREL = 4

NTILES = 16
NCORES = 2
NPAD = 10240
CHUNK = 5120
EPT = N_EDGES // NTILES
SB = 2000
NSB = EPT // SB
BE = 80
NB = SB // BE
DA = 160
ACC_ROWS = NREL * CHUNK + 128
DUMP = NREL * CHUNK
ZROWS = ACC_ROWS // NTILES
RPT = NREL * CHUNK // NTILES


def _sc_body(x_hbm, src_hbm, dst_hbm, typ_hbm, zrow_hbm,
             sums_hbm,
             src_sb, dst_sb, typ_sb, srow, arow, rows0, rows1, rows2,
             gs0, gs1, gs2, ss0, ss1, ss2,
             acc_sh):
    core = lax.axis_index("c")
    sub = lax.axis_index("s")
    rows = (rows0, rows1, rows2)
    gsem = (gs0, gs1, gs2)
    ssem = (ss0, ss1, ss2)

    if True:
        lo = core * CHUNK

        pltpu.sync_copy(zrow_hbm, acc_sh.at[pl.ds(sub * ZROWS, ZROWS)])
        plsc.subcore_barrier()

        @pl.loop(0, NSB)
        def _superblock(sb):
            base = sub * EPT + sb * SB
            pltpu.sync_copy(src_hbm.at[pl.ds(base, SB)], src_sb)
            pltpu.sync_copy(dst_hbm.at[pl.ds(base, SB)], dst_sb)
            pltpu.sync_copy(typ_hbm.at[pl.ds(base, SB)], typ_sb)

            def prep(b):
                slot = b % 3
                for g in range(BE // 16):
                    off = b * BE + g * 16
                    d16 = dst_sb[pl.ds(off, 16)]
                    t16 = typ_sb[pl.ds(off, 16)]
                    s16 = src_sb[pl.ds(off, 16)]
                    m = (d16 >= lo) & (d16 < lo + CHUNK)
                    a16 = jnp.where(m, t16 * CHUNK + (d16 - lo), DUMP)
                    srow[slot, pl.ds(g * 16, 16)] = s16
                    arow[slot, pl.ds(g * 16, 16)] = a16

            gds = [None] * NB
            sds = [None] * NB
            prep(0)
            gds[0] = pltpu.async_copy(x_hbm.at[srow.at[0]], rows[0], gsem[0])
            for b in range(NB):
                cur = b % 3
                if b + 1 < NB:
                    nxt = (b + 1) % 3
                    if b >= 2:
                        sds[b - 2].wait()
                    prep(b + 1)
                    gds[b + 1] = pltpu.async_copy(
                        x_hbm.at[srow.at[nxt]], rows[nxt], gsem[nxt])
                gds[b].wait()
                sds[b] = pltpu.async_copy(
                    rows[cur], acc_sh.at[arow.at[cur]], ssem[cur], add=True)
            sds[NB - 3].wait()
            sds[NB - 2].wait()
            sds[NB - 1].wait()

        plsc.subcore_barrier()

        rel = sub // (CHUNK // RPT)
        q = sub % (CHUNK // RPT)
        row0 = rel * NPAD + lo + q * RPT
        pltpu.sync_copy(acc_sh.at[pl.ds(sub * RPT, RPT)],
                        sums_hbm.at[pl.ds(row0, RPT)])


def _sc_aggregate(xa, src, dst, etype):
    zrow = jnp.zeros((ZROWS, DA), jnp.bfloat16)
    mesh = plsc.VectorSubcoreMesh(core_axis_name="c", subcore_axis_name="s")
    f = pl.kernel(
        _sc_body,
        out_type=jax.ShapeDtypeStruct((NREL * NPAD, DA), jnp.bfloat16),
        mesh=mesh,
        compiler_params=pltpu.CompilerParams(use_tc_tiling_on_sc=False),
        scratch_types=[
            pltpu.VMEM((SB,), jnp.int32),
            pltpu.VMEM((SB,), jnp.int32),
            pltpu.VMEM((SB,), jnp.int32),
            pltpu.VMEM((3, BE), jnp.int32),
            pltpu.VMEM((3, BE), jnp.int32),
            pltpu.VMEM((BE, DA), jnp.bfloat16),
            pltpu.VMEM((BE, DA), jnp.bfloat16),
            pltpu.VMEM((BE, DA), jnp.bfloat16),
            pltpu.SemaphoreType.DMA,
            pltpu.SemaphoreType.DMA,
            pltpu.SemaphoreType.DMA,
            pltpu.SemaphoreType.DMA,
            pltpu.SemaphoreType.DMA,
            pltpu.SemaphoreType.DMA,
            pltpu.VMEM_SHARED((ACC_ROWS, DA), jnp.bfloat16),
        ],
    )
    return f(xa, src, dst, etype, zrow)


BM = 400


def _tc_body(s_ref, w_ref, b_ref, o_ref):
    acc = jnp.zeros((BM, D), jnp.float32)
    for r in range(NREL):
        blk = s_ref[r].astype(jnp.float32)
        s = blk[:, :D]
        cr = blk[:, D:D + 1]
        t = jnp.dot(s, w_ref[r], precision=lax.Precision.HIGHEST,
                    preferred_element_type=jnp.float32)
        acc = acc + jnp.where(cr > 0.0, t / jnp.maximum(cr, 1.0),
                              jnp.zeros_like(t))
    o_ref[...] = acc + b_ref[...]


def _tc_combine(sums, weights, bias):
    grid = (N_NODES // BM,)
    return pl.pallas_call(
        _tc_body,
        grid=grid,
        in_specs=[
            pl.BlockSpec((NREL, BM, DA), lambda i: (0, i, 0)),
            pl.BlockSpec((NREL, D, D), lambda i: (0, 0, 0)),
            pl.BlockSpec((1, D), lambda i: (0, 0)),
        ],
        out_specs=pl.BlockSpec((BM, D), lambda i: (i, 0)),
        out_shape=jax.ShapeDtypeStruct((N_NODES, D), jnp.float32),
    )(sums, weights, bias)


@jax.jit
def kernel(x, edge_index, edge_type, weight_matrices, bias):
    src = edge_index[0]
    dst = edge_index[1]
    ones_col = jnp.ones((N_NODES, 1), jnp.float32)
    pad = jnp.zeros((N_NODES, DA - D - 1), jnp.float32)
    xa = jnp.concatenate([x, ones_col, pad], axis=1).astype(jnp.bfloat16)
    sums = _sc_aggregate(xa, src, dst, edge_type)
    sums = sums.reshape(NREL, NPAD, DA)
    return _tc_combine(sums, weight_matrices, bias.reshape(1, D))


# speedup vs baseline: 19.2945x; 1.0225x over previous
import functools

import jax
import jax.numpy as jnp
from jax import lax
from jax.experimental import pallas as pl
from jax.experimental.pallas import tpu as pltpu
from jax.experimental.pallas import tpu_sc as plsc

N_NODES = 10000
N_EDGES = 320000
D = 128
NREL = 4

NTILES = 16
NCORES = 2
NPAD = 10240
CHUNK = 5120
EPT = N_EDGES // NTILES
SB = 2000
NSB = EPT // SB
BE = 80
NB = SB // BE
DA = 160
ACC_ROWS = NREL * CHUNK + 128
DUMP = NREL * CHUNK
ZROWS = ACC_ROWS // NTILES
RPT = NREL * CHUNK // NTILES


def _sc_body(x_hbm, src_hbm, dst_hbm, typ_hbm, zrow_hbm,
             main_hbm, aux_hbm,
             src_sb, dst_sb, typ_sb, srow, arow, rows0, rows1, rows2,
             gs0, gs1, gs2, ss0, ss1, ss2,
             acc_sh):
    core = lax.axis_index("c")
    sub = lax.axis_index("s")
    rows = (rows0, rows1, rows2)
    gsem = (gs0, gs1, gs2)
    ssem = (ss0, ss1, ss2)

    if True:
        lo = core * CHUNK

        pltpu.sync_copy(zrow_hbm, acc_sh.at[pl.ds(sub * ZROWS, ZROWS)])
        plsc.subcore_barrier()

        @pl.loop(0, NSB)
        def _superblock(sb):
            base = sub * EPT + sb * SB
            pltpu.sync_copy(src_hbm.at[pl.ds(base, SB)], src_sb)
            pltpu.sync_copy(dst_hbm.at[pl.ds(base, SB)], dst_sb)
            pltpu.sync_copy(typ_hbm.at[pl.ds(base, SB)], typ_sb)

            def prep(b):
                slot = b % 3
                for g in range(BE // 16):
                    off = b * BE + g * 16
                    d16 = dst_sb[pl.ds(off, 16)]
                    t16 = typ_sb[pl.ds(off, 16)]
                    s16 = src_sb[pl.ds(off, 16)]
                    m = (d16 >= lo) & (d16 < lo + CHUNK)
                    a16 = jnp.where(m, t16 * CHUNK + (d16 - lo), DUMP)
                    srow[slot, pl.ds(g * 16, 16)] = s16
                    arow[slot, pl.ds(g * 16, 16)] = a16

            gds = [None] * NB
            sds = [None] * NB
            prep(0)
            gds[0] = pltpu.async_copy(x_hbm.at[srow.at[0]], rows[0], gsem[0])
            for b in range(NB):
                cur = b % 3
                if b + 1 < NB:
                    nxt = (b + 1) % 3
                    if b >= 2:
                        sds[b - 2].wait()
                    prep(b + 1)
                    gds[b + 1] = pltpu.async_copy(
                        x_hbm.at[srow.at[nxt]], rows[nxt], gsem[nxt])
                gds[b].wait()
                sds[b] = pltpu.async_copy(
                    rows[cur], acc_sh.at[arow.at[cur]], ssem[cur], add=True)
            sds[NB - 3].wait()
            sds[NB - 2].wait()
            sds[NB - 1].wait()

        plsc.subcore_barrier()

        rel = sub // (CHUNK // RPT)
        q = sub % (CHUNK // RPT)
        row0 = rel * NPAD + lo + q * RPT
        pltpu.sync_copy(acc_sh.at[pl.ds(sub * RPT, RPT), pl.ds(0, D)],
                        main_hbm.at[pl.ds(row0, RPT)])
        pltpu.sync_copy(acc_sh.at[pl.ds(sub * RPT, RPT), pl.ds(D, DA - D)],
                        aux_hbm.at[pl.ds(row0, RPT)])


def _sc_aggregate(xa, src, dst, etype):
    zrow = jnp.zeros((ZROWS, DA), jnp.bfloat16)
    mesh = plsc.VectorSubcoreMesh(core_axis_name="c", subcore_axis_name="s")
    f = pl.kernel(
        _sc_body,
        out_type=(
            jax.ShapeDtypeStruct((NREL * NPAD, D), jnp.bfloat16),
            jax.ShapeDtypeStruct((NREL * NPAD, DA - D), jnp.bfloat16),
        ),
        mesh=mesh,
        compiler_params=pltpu.CompilerParams(use_tc_tiling_on_sc=False),
        scratch_types=[
            pltpu.VMEM((SB,), jnp.int32),
            pltpu.VMEM((SB,), jnp.int32),
            pltpu.VMEM((SB,), jnp.int32),
            pltpu.VMEM((3, BE), jnp.int32),
            pltpu.VMEM((3, BE), jnp.int32),
            pltpu.VMEM((BE, DA), jnp.bfloat16),
            pltpu.VMEM((BE, DA), jnp.bfloat16),
            pltpu.VMEM((BE, DA), jnp.bfloat16),
            pltpu.SemaphoreType.DMA,
            pltpu.SemaphoreType.DMA,
            pltpu.SemaphoreType.DMA,
            pltpu.SemaphoreType.DMA,
            pltpu.SemaphoreType.DMA,
            pltpu.SemaphoreType.DMA,
            pltpu.VMEM_SHARED((ACC_ROWS, DA), jnp.bfloat16),
        ],
    )
    return f(xa, src, dst, etype, zrow)


BM = 400


def _tc_body(s_ref, c_ref, w_ref, b_ref, o_ref):
    acc = jnp.zeros((BM, D), jnp.float32)
    for r in range(NREL):
        s = s_ref[r].astype(jnp.float32)
        cr = c_ref[r][:, 0:1].astype(jnp.float32)
        t = jnp.dot(s, w_ref[r], precision=lax.Precision.HIGHEST,
                    preferred_element_type=jnp.float32)
        acc = acc + jnp.where(cr > 0.0, t / jnp.maximum(cr, 1.0),
                              jnp.zeros_like(t))
    o_ref[...] = acc + b_ref[...]


def _tc_combine(sums, cnts, weights, bias):
    grid = (N_NODES // BM,)
    return pl.pallas_call(
        _tc_body,
        grid=grid,
        in_specs=[
            pl.BlockSpec((NREL, BM, D), lambda i: (0, i, 0)),
            pl.BlockSpec((NREL, BM, DA - D), lambda i: (0, i, 0)),
            pl.BlockSpec((NREL, D, D), lambda i: (0, 0, 0)),
            pl.BlockSpec((1, D), lambda i: (0, 0)),
        ],
        out_specs=pl.BlockSpec((BM, D), lambda i: (i, 0)),
        out_shape=jax.ShapeDtypeStruct((N_NODES, D), jnp.float32),
    )(sums, cnts, weights, bias)


@jax.jit
def kernel(x, edge_index, edge_type, weight_matrices, bias):
    src = edge_index[0]
    dst = edge_index[1]
    ones_col = jnp.ones((N_NODES, 1), jnp.float32)
    pad = jnp.zeros((N_NODES, DA - D - 1), jnp.float32)
    xa = jnp.concatenate([x, ones_col, pad], axis=1).astype(jnp.bfloat16)
    sums, cnts = _sc_aggregate(xa, src, dst, edge_type)
    sums = sums.reshape(NREL, NPAD, D)
    cnts = cnts.reshape(NREL, NPAD, DA - D)
    return _tc_combine(sums, cnts, weight_matrices, bias.reshape(1, D))
